# baseline (speedup 1.0000x reference)
import functools

import jax
import jax.numpy as jnp
from jax import lax
from jax.experimental import pallas as pl
from jax.experimental.pallas import tpu as pltpu
from jax.experimental.pallas import tpu_sc as plsc

_NUM_CORES = 2
_NUM_SUBCORES = 16
_NUM_WORKERS = _NUM_CORES * _NUM_SUBCORES
_LANES = 16
_WIN = 128
_CHUNK = 8
_SC_SHARE = 8192
_SB = 256


def _sc_gather(tT, idx, n):
    dim = tT.shape[0]
    b_per_w = n // _NUM_WORKERS
    mesh = plsc.VectorSubcoreMesh(core_axis_name="c", subcore_axis_name="s")

    @functools.partial(
        pl.kernel, mesh=mesh,
        out_type=jax.ShapeDtypeStruct((dim, n), tT.dtype),
        scratch_types=[
            pltpu.VMEM((b_per_w,), jnp.int32),
            pltpu.VMEM((2 * _CHUNK, dim, _WIN), tT.dtype),
            pltpu.VMEM((dim, b_per_w), tT.dtype),
            pltpu.SemaphoreType.DMA,
            pltpu.SemaphoreType.DMA,
        ],
        compiler_params=pltpu.CompilerParams(disable_bounds_checks=True,
                                             needs_layout_passes=False),
    )
    def gather_kernel(tT_hbm, idx_hbm, outT_hbm, idx_v, win_v, acc_v,
                      sem0, sem1):
        wid = lax.axis_index("s") * _NUM_CORES + lax.axis_index("c")
        base = wid * b_per_w
        pltpu.sync_copy(idx_hbm.at[pl.ds(base, b_per_w)], idx_v)

        row_lo = lax.iota(jnp.int32, _LANES)
        row_hi = row_lo + _LANES
        sems = (sem0, sem1)

        def fire(v, half):
            for m in range(_CHUNK):
                s = v[half * _CHUNK + m]
                soff = pl.multiple_of((s >> 7) << 7, _WIN)
                pltpu.async_copy(
                    tT_hbm.at[:, pl.ds(soff, _WIN)],
                    win_v.at[half * _CHUNK + m], sems[half])

        def drain_and_extract(v, i, half):
            for m in range(_CHUNK):
                pltpu.make_async_copy(
                    tT_hbm.at[:, pl.ds(0, _WIN)],
                    win_v.at[half * _CHUNK + m], sems[half]).wait()
            for m in range(_CHUNK):
                c = v[half * _CHUNK + m] & (_WIN - 1)
                cvec = jnp.zeros((_LANES,), jnp.int32) + c
                kvec = jnp.zeros((_LANES,), jnp.int32) + (i + half * _CHUNK + m)
                g0 = plsc.load_gather(win_v.at[half * _CHUNK + m],
                                      [row_lo, cvec])
                g1 = plsc.load_gather(win_v.at[half * _CHUNK + m],
                                      [row_hi, cvec])
                plsc.store_scatter(acc_v, [row_lo, kvec], g0)
                plsc.store_scatter(acc_v, [row_hi, kvec], g1)

        v0 = idx_v[pl.ds(0, 2 * _CHUNK)]
        fire(v0, 0)

        @pl.loop(0, b_per_w, step=2 * _CHUNK)
        def _(i):
            v = idx_v[pl.ds(i, 2 * _CHUNK)]
            fire(v, 1)
            drain_and_extract(v, i, 0)

            @pl.when(i + 2 * _CHUNK < b_per_w)
            def _():
                vn = idx_v[pl.ds(i + 2 * _CHUNK, 2 * _CHUNK)]
                fire(vn, 0)

            drain_and_extract(v, i, 1)

        pltpu.sync_copy(acc_v, outT_hbm.at[:, pl.ds(base, b_per_w)])

    return gather_kernel(tT, idx)


def _tc_gather(tT, idx, n):
    dim = tT.shape[0]
    n_steps = n // _SB
    idx2d = idx.reshape(n, 1)

    def body(idx_pf, tT_any, idx_blk, out_ref, win0, win1, sem0, sem1):
        g = pl.program_id(0)

        def fire(step, win, sem):
            base = step * _SB

            def body_m(m, carry):
                s = idx_pf[base + m]
                soff = pl.multiple_of((s >> 7) << 7, _WIN)
                pltpu.make_async_copy(
                    tT_any.at[:, pl.ds(soff, _WIN)], win.at[m], sem).start()
                return carry

            lax.fori_loop(0, _SB, body_m, 0, unroll=16)

        def drain(win, sem):
            def body_m(m, carry):
                pltpu.make_async_copy(
                    tT_any.at[:, pl.ds(0, _WIN)], win.at[m], sem).wait()
                return carry

            lax.fori_loop(0, _SB, body_m, 0, unroll=16)

        def extract(win):
            w = win[...]
            c = (idx_blk[...] & (_WIN - 1))[:, :, None]
            cfull = jnp.broadcast_to(c, (_SB, dim, 1))
            out_ref[...] = jnp.take_along_axis(w, cfull, axis=2)[..., 0]

        @pl.when(g == 0)
        def _():
            fire(0, win0, sem0)

        even = g % 2 == 0

        @pl.when((g + 1 < n_steps) & even)
        def _():
            fire(g + 1, win1, sem1)

        @pl.when((g + 1 < n_steps) & jnp.logical_not(even))
        def _():
            fire(g + 1, win0, sem0)

        @pl.when(even)
        def _():
            drain(win0, sem0)
            extract(win0)

        @pl.when(jnp.logical_not(even))
        def _():
            drain(win1, sem1)
            extract(win1)

    grid_spec = pltpu.PrefetchScalarGridSpec(
        num_scalar_prefetch=1,
        grid=(n_steps,),
        in_specs=[
            pl.BlockSpec(memory_space=pl.ANY),
            pl.BlockSpec((_SB, 1), lambda g, idx_ref: (g, 0)),
        ],
        out_specs=pl.BlockSpec((_SB, dim), lambda g, idx_ref: (g, 0)),
        scratch_shapes=[
            pltpu.VMEM((_SB, dim, _WIN), tT.dtype),
            pltpu.VMEM((_SB, dim, _WIN), tT.dtype),
            pltpu.SemaphoreType.DMA,
            pltpu.SemaphoreType.DMA,
        ],
    )
    return pl.pallas_call(
        body,
        grid_spec=grid_spec,
        out_shape=jax.ShapeDtypeStruct((n, dim), tT.dtype),
        compiler_params=pltpu.CompilerParams(disable_bounds_checks=True),
    )(idx, tT, idx2d)


def kernel(user_indices, table):
    batch = user_indices.shape[0]
    idx = user_indices.astype(jnp.int32)
    tT = table.T

    outT_sc = _sc_gather(tT, idx[:_SC_SHARE], _SC_SHARE)
    out_tc = _tc_gather(tT, idx[_SC_SHARE:], batch - _SC_SHARE)
    return jnp.concatenate([outT_sc.T, out_tc], axis=0)


# speedup vs baseline: 1.3133x; 1.3133x over previous
import functools

import jax
import jax.numpy as jnp
from jax import lax
from jax.experimental import pallas as pl
from jax.experimental.pallas import tpu as pltpu
from jax.experimental.pallas import tpu_sc as plsc

_NUM_CORES = 2
_NUM_SUBCORES = 16
_NUM_WORKERS = _NUM_CORES * _NUM_SUBCORES
_LANES = 16
_WIN = 128
_CHUNK = 8
_SC_SHARE = 8192
_SB = 256


def _sc_gather(tT, idx, n):
    dim = tT.shape[0]
    b_per_w = n // _NUM_WORKERS
    mesh = plsc.VectorSubcoreMesh(core_axis_name="c", subcore_axis_name="s")

    @functools.partial(
        pl.kernel, mesh=mesh,
        out_type=jax.ShapeDtypeStruct((dim, n), tT.dtype),
        scratch_types=[
            pltpu.VMEM((b_per_w,), jnp.int32),
            pltpu.VMEM((2 * _CHUNK, dim, _WIN), tT.dtype),
            pltpu.VMEM((dim, b_per_w), tT.dtype),
            pltpu.SemaphoreType.DMA,
            pltpu.SemaphoreType.DMA,
        ],
        compiler_params=pltpu.CompilerParams(disable_bounds_checks=True,
                                             needs_layout_passes=False),
    )
    def gather_kernel(tT_hbm, idx_hbm, outT_hbm, idx_v, win_v, acc_v,
                      sem0, sem1):
        wid = lax.axis_index("s") * _NUM_CORES + lax.axis_index("c")
        base = wid * b_per_w
        pltpu.sync_copy(idx_hbm.at[pl.ds(base, b_per_w)], idx_v)

        row_lo = lax.iota(jnp.int32, _LANES)
        row_hi = row_lo + _LANES
        sems = (sem0, sem1)

        def fire(v, half):
            for m in range(_CHUNK):
                s = v[half * _CHUNK + m]
                soff = pl.multiple_of((s >> 7) << 7, _WIN)
                pltpu.async_copy(
                    tT_hbm.at[:, pl.ds(soff, _WIN)],
                    win_v.at[half * _CHUNK + m], sems[half])

        def drain_and_extract(v, i, half):
            for m in range(_CHUNK):
                pltpu.make_async_copy(
                    tT_hbm.at[:, pl.ds(0, _WIN)],
                    win_v.at[half * _CHUNK + m], sems[half]).wait()
            for m in range(_CHUNK):
                c = v[half * _CHUNK + m] & (_WIN - 1)
                cvec = jnp.zeros((_LANES,), jnp.int32) + c
                kvec = jnp.zeros((_LANES,), jnp.int32) + (i + half * _CHUNK + m)
                g0 = plsc.load_gather(win_v.at[half * _CHUNK + m],
                                      [row_lo, cvec])
                g1 = plsc.load_gather(win_v.at[half * _CHUNK + m],
                                      [row_hi, cvec])
                plsc.store_scatter(acc_v, [row_lo, kvec], g0)
                plsc.store_scatter(acc_v, [row_hi, kvec], g1)

        v0 = idx_v[pl.ds(0, 2 * _CHUNK)]
        fire(v0, 0)

        @pl.loop(0, b_per_w, step=2 * _CHUNK)
        def _(i):
            v = idx_v[pl.ds(i, 2 * _CHUNK)]
            fire(v, 1)
            drain_and_extract(v, i, 0)

            @pl.when(i + 2 * _CHUNK < b_per_w)
            def _():
                vn = idx_v[pl.ds(i + 2 * _CHUNK, 2 * _CHUNK)]
                fire(vn, 0)

            drain_and_extract(v, i, 1)

        pltpu.sync_copy(acc_v, outT_hbm.at[:, pl.ds(base, b_per_w)])

    return gather_kernel(tT, idx)


def _tc_gather(tT, idx, n):
    dim = tT.shape[0]
    n_steps = n // _SB
    idx2d = idx.reshape(n, 1)

    def body(idx_pf, tT_any, idx_blk, out_ref, win0, win1, sem0, sem1):
        g = pl.program_id(0)

        def fire(step, win, sem):
            base = step * _SB

            def body_m(m, carry):
                s = idx_pf[base + m]
                soff = pl.multiple_of((s >> 7) << 7, _WIN)
                pltpu.make_async_copy(
                    tT_any.at[:, pl.ds(soff, _WIN)], win.at[m], sem).start()
                return carry

            lax.fori_loop(0, _SB, body_m, 0, unroll=16)

        def drain(win, sem):
            def body_m(m, carry):
                pltpu.make_async_copy(
                    tT_any.at[:, pl.ds(0, _WIN)], win.at[m], sem).wait()
                return carry

            lax.fori_loop(0, _SB, body_m, 0, unroll=16)

        def extract(win):
            w = win[...]
            c = (idx_blk[...] & (_WIN - 1))[:, :, None]
            lane = lax.broadcasted_iota(jnp.int32, (_SB, dim, _WIN), 2)
            out_ref[...] = jnp.sum(
                jnp.where(lane == c, w, 0.0), axis=2)

        @pl.when(g == 0)
        def _():
            fire(0, win0, sem0)

        even = g % 2 == 0

        @pl.when((g + 1 < n_steps) & even)
        def _():
            fire(g + 1, win1, sem1)

        @pl.when((g + 1 < n_steps) & jnp.logical_not(even))
        def _():
            fire(g + 1, win0, sem0)

        @pl.when(even)
        def _():
            drain(win0, sem0)
            extract(win0)

        @pl.when(jnp.logical_not(even))
        def _():
            drain(win1, sem1)
            extract(win1)

    grid_spec = pltpu.PrefetchScalarGridSpec(
        num_scalar_prefetch=1,
        grid=(n_steps,),
        in_specs=[
            pl.BlockSpec(memory_space=pl.ANY),
            pl.BlockSpec((_SB, 1), lambda g, idx_ref: (g, 0)),
        ],
        out_specs=pl.BlockSpec((_SB, dim), lambda g, idx_ref: (g, 0)),
        scratch_shapes=[
            pltpu.VMEM((_SB, dim, _WIN), tT.dtype),
            pltpu.VMEM((_SB, dim, _WIN), tT.dtype),
            pltpu.SemaphoreType.DMA,
            pltpu.SemaphoreType.DMA,
        ],
    )
    return pl.pallas_call(
        body,
        grid_spec=grid_spec,
        out_shape=jax.ShapeDtypeStruct((n, dim), tT.dtype),
        compiler_params=pltpu.CompilerParams(disable_bounds_checks=True),
    )(idx, tT, idx2d)


def kernel(user_indices, table):
    batch = user_indices.shape[0]
    idx = user_indices.astype(jnp.int32)
    tT = table.T

    outT_sc = _sc_gather(tT, idx[:_SC_SHARE], _SC_SHARE)
    out_tc = _tc_gather(tT, idx[_SC_SHARE:], batch - _SC_SHARE)
    return jnp.concatenate([outT_sc.T, out_tc], axis=0)


# speedup vs baseline: 1.3363x; 1.0175x over previous
import functools

import jax
import jax.numpy as jnp
from jax import lax
from jax.experimental import pallas as pl
from jax.experimental.pallas import tpu as pltpu
from jax.experimental.pallas import tpu_sc as plsc

_NUM_CORES = 2
_NUM_SUBCORES = 16
_NUM_WORKERS = _NUM_CORES * _NUM_SUBCORES
_LANES = 16
_WIN = 128
_CHUNK = 8
_SC_SHARE = 8192
_SB = 512


def _sc_gather(tT, idx, n):
    dim = tT.shape[0]
    b_per_w = n // _NUM_WORKERS
    mesh = plsc.VectorSubcoreMesh(core_axis_name="c", subcore_axis_name="s")

    @functools.partial(
        pl.kernel, mesh=mesh,
        out_type=jax.ShapeDtypeStruct((dim, n), tT.dtype),
        scratch_types=[
            pltpu.VMEM((b_per_w,), jnp.int32),
            pltpu.VMEM((2 * _CHUNK, dim, _WIN), tT.dtype),
            pltpu.VMEM((dim, b_per_w), tT.dtype),
            pltpu.SemaphoreType.DMA,
            pltpu.SemaphoreType.DMA,
        ],
        compiler_params=pltpu.CompilerParams(disable_bounds_checks=True,
                                             needs_layout_passes=False),
    )
    def gather_kernel(tT_hbm, idx_hbm, outT_hbm, idx_v, win_v, acc_v,
                      sem0, sem1):
        wid = lax.axis_index("s") * _NUM_CORES + lax.axis_index("c")
        base = wid * b_per_w
        pltpu.sync_copy(idx_hbm.at[pl.ds(base, b_per_w)], idx_v)

        row_lo = lax.iota(jnp.int32, _LANES)
        row_hi = row_lo + _LANES
        sems = (sem0, sem1)

        def fire(v, half):
            for m in range(_CHUNK):
                s = v[half * _CHUNK + m]
                soff = pl.multiple_of((s >> 7) << 7, _WIN)
                pltpu.async_copy(
                    tT_hbm.at[:, pl.ds(soff, _WIN)],
                    win_v.at[half * _CHUNK + m], sems[half])

        def drain_and_extract(v, i, half):
            for m in range(_CHUNK):
                pltpu.make_async_copy(
                    tT_hbm.at[:, pl.ds(0, _WIN)],
                    win_v.at[half * _CHUNK + m], sems[half]).wait()
            for m in range(_CHUNK):
                c = v[half * _CHUNK + m] & (_WIN - 1)
                cvec = jnp.zeros((_LANES,), jnp.int32) + c
                kvec = jnp.zeros((_LANES,), jnp.int32) + (i + half * _CHUNK + m)
                g0 = plsc.load_gather(win_v.at[half * _CHUNK + m],
                                      [row_lo, cvec])
                g1 = plsc.load_gather(win_v.at[half * _CHUNK + m],
                                      [row_hi, cvec])
                plsc.store_scatter(acc_v, [row_lo, kvec], g0)
                plsc.store_scatter(acc_v, [row_hi, kvec], g1)

        v0 = idx_v[pl.ds(0, 2 * _CHUNK)]
        fire(v0, 0)

        @pl.loop(0, b_per_w, step=2 * _CHUNK)
        def _(i):
            v = idx_v[pl.ds(i, 2 * _CHUNK)]
            fire(v, 1)
            drain_and_extract(v, i, 0)

            @pl.when(i + 2 * _CHUNK < b_per_w)
            def _():
                vn = idx_v[pl.ds(i + 2 * _CHUNK, 2 * _CHUNK)]
                fire(vn, 0)

            drain_and_extract(v, i, 1)

        pltpu.sync_copy(acc_v, outT_hbm.at[:, pl.ds(base, b_per_w)])

    return gather_kernel(tT, idx)


def _tc_gather(tT, idx, n):
    dim = tT.shape[0]
    n_steps = n // _SB
    idx2d = idx.reshape(n, 1)

    def body(idx_pf, tT_any, idx_blk, out_ref, win0, win1, sem0, sem1):
        g = pl.program_id(0)

        def fire(step, win, sem):
            base = step * _SB

            def body_m(m, carry):
                s = idx_pf[base + m]
                soff = pl.multiple_of((s >> 7) << 7, _WIN)
                pltpu.make_async_copy(
                    tT_any.at[:, pl.ds(soff, _WIN)], win.at[m], sem).start()
                return carry

            lax.fori_loop(0, _SB, body_m, 0, unroll=16)

        def drain(win, sem):
            def body_m(m, carry):
                pltpu.make_async_copy(
                    tT_any.at[:, pl.ds(0, _WIN)], win.at[m], sem).wait()
                return carry

            lax.fori_loop(0, _SB, body_m, 0, unroll=16)

        def extract(win):
            w = win[...]
            c = (idx_blk[...] & (_WIN - 1))[:, :, None]
            lane = lax.broadcasted_iota(jnp.int32, (_SB, dim, _WIN), 2)
            out_ref[...] = jnp.sum(
                jnp.where(lane == c, w, 0.0), axis=2)

        @pl.when(g == 0)
        def _():
            fire(0, win0, sem0)

        even = g % 2 == 0

        @pl.when((g + 1 < n_steps) & even)
        def _():
            fire(g + 1, win1, sem1)

        @pl.when((g + 1 < n_steps) & jnp.logical_not(even))
        def _():
            fire(g + 1, win0, sem0)

        @pl.when(even)
        def _():
            drain(win0, sem0)
            extract(win0)

        @pl.when(jnp.logical_not(even))
        def _():
            drain(win1, sem1)
            extract(win1)

    grid_spec = pltpu.PrefetchScalarGridSpec(
        num_scalar_prefetch=1,
        grid=(n_steps,),
        in_specs=[
            pl.BlockSpec(memory_space=pl.ANY),
            pl.BlockSpec((_SB, 1), lambda g, idx_ref: (g, 0)),
        ],
        out_specs=pl.BlockSpec((_SB, dim), lambda g, idx_ref: (g, 0)),
        scratch_shapes=[
            pltpu.VMEM((_SB, dim, _WIN), tT.dtype),
            pltpu.VMEM((_SB, dim, _WIN), tT.dtype),
            pltpu.SemaphoreType.DMA,
            pltpu.SemaphoreType.DMA,
        ],
    )
    return pl.pallas_call(
        body,
        grid_spec=grid_spec,
        out_shape=jax.ShapeDtypeStruct((n, dim), tT.dtype),
        compiler_params=pltpu.CompilerParams(
            disable_bounds_checks=True,
            vmem_limit_bytes=60 * 1024 * 1024),
    )(idx, tT, idx2d)


def kernel(user_indices, table):
    batch = user_indices.shape[0]
    idx = user_indices.astype(jnp.int32)
    tT = table.T

    outT_sc = _sc_gather(tT, idx[:_SC_SHARE], _SC_SHARE)
    out_tc = _tc_gather(tT, idx[_SC_SHARE:], batch - _SC_SHARE)
    return jnp.concatenate([outT_sc.T, out_tc], axis=0)
